# baseline (speedup 1.0000x reference)
import functools

import numpy as np

import jax
import jax.numpy as jnp
from jax import lax
from jax.experimental import pallas as pl
from jax.experimental.pallas import tpu as pltpu
from jax.experimental.pallas import tpu_sc as plsc

N_LAYER = 3
EMB = 128
D_EDGE = 4
N = 10000
E = 320000
G = 64

NC, NS = 2, 16
NW = NC * NS
CH = 128
EPAD = 327680
NCHUNK = EPAD // CH
CPW = NCHUNK // NW
NPAD = 10240
RPS = NPAD // NS

BN = 1000
BE = 3200


def _sc_mesh():
    return plsc.VectorSubcoreMesh(core_axis_name="c", subcore_axis_name="s")


def _sc_msg(table, e, src2d):
    TPS = 632
    TLAST = N - (NS - 1) * TPS

    @functools.partial(
        pl.kernel,
        mesh=_sc_mesh(),
        out_type=jax.ShapeDtypeStruct((EPAD, EMB), jnp.float32),
        scratch_types=[
            pltpu.VMEM((8, CH), jnp.int32),
            pltpu.VMEM((CH, EMB), jnp.float32),
            pltpu.VMEM((CH, EMB), jnp.float32),
            pltpu.VMEM((CH, EMB), jnp.float32),
            pltpu.VMEM_SHARED((N, EMB), jnp.float32),
            pltpu.SemaphoreType.DMA,
            pltpu.SemaphoreType.DMA,
            pltpu.SemaphoreType.DMA,
            pltpu.SemaphoreType.DMA,
            pltpu.SemaphoreType.DMA,
        ],
    )
    def k(table_hbm, e_hbm, src_hbm, out_hbm, ring, av, e0, e1, tbl,
          semr, seme0, seme1, semo0, semo1):
        c = lax.axis_index("c")
        s = lax.axis_index("s")
        wid = s * NC + c
        base = wid * CPW

        toff = pl.multiple_of(s * TPS, 8)

        @pl.when(s < NS - 1)
        def _():
            pltpu.sync_copy(table_hbm.at[pl.ds(toff, TPS)],
                            tbl.at[pl.ds(toff, TPS)])

        @pl.when(s == NS - 1)
        def _():
            pltpu.sync_copy(table_hbm.at[pl.ds((NS - 1) * TPS, TLAST)],
                            tbl.at[pl.ds((NS - 1) * TPS, TLAST)])

        pltpu.async_copy(src_hbm.at[pl.ds(base, 8)], ring, semr)
        pltpu.async_copy(e_hbm.at[pl.ds(base * CH, CH)], e0, seme0)
        plsc.subcore_barrier()

        bufs = ((e0, seme0, semo0), (e1, seme1, semo1))

        def half(j, q, which):
            eb, seme, semo = bufs[which]
            eo, semeo, semoo = bufs[1 - which]
            pltpu.sync_copy(tbl.at[ring.at[q]], av)
            if q == 7:
                @pl.when(j + 1 < CPW)
                def _():
                    roff = pl.multiple_of(base + j + 1, 8)
                    pltpu.async_copy(src_hbm.at[pl.ds(roff, 8)], ring, semr)

            @pl.when(j + 1 < CPW)
            def _():
                @pl.when(j >= 1)
                def _():
                    pltpu.make_async_copy(out_hbm.at[pl.ds(0, CH)], eo,
                                          semoo).wait()
                pltpu.async_copy(e_hbm.at[pl.ds((base + j + 1) * CH, CH)],
                                 eo, semeo)

            pltpu.make_async_copy(e_hbm.at[pl.ds(0, CH)], eb, seme).wait()

            @pl.loop(0, CH)
            def _(r):
                for cc in range(EMB // 16):
                    sl = pl.ds(cc * 16, 16)
                    eb[r, sl] = jnp.maximum(av[r, sl] + eb[r, sl], 0.0)

            pltpu.async_copy(eb, out_hbm.at[pl.ds((base + j) * CH, CH)], semo)

        @pl.loop(0, CPW, step=8)
        def _(j):
            pltpu.make_async_copy(src_hbm.at[pl.ds(0, 8)], ring, semr).wait()
            for q in range(8):
                half(j + q, q, q % 2)

        pltpu.make_async_copy(out_hbm.at[pl.ds(0, CH)], e0, semo0).wait()
        pltpu.make_async_copy(out_hbm.at[pl.ds(0, CH)], e1, semo1).wait()

    return k(table, e, src2d)


def _sc_scatter_add(msg, dst2d, zeros):
    @functools.partial(
        pl.kernel,
        mesh=_sc_mesh(),
        out_type=jax.ShapeDtypeStruct((NC, NPAD, EMB), jnp.float32),
        scratch_types=[
            pltpu.VMEM((CPW, CH), jnp.int32),
            pltpu.VMEM((CH, EMB), jnp.float32),
            pltpu.VMEM((CH, EMB), jnp.float32),
            pltpu.VMEM_SHARED((NPAD + 8, EMB), jnp.float32),
            pltpu.SemaphoreType.DMA,
            pltpu.SemaphoreType.DMA,
        ],
    )
    def k(msg_hbm, dst_hbm, z_hbm, out_hbm, dst_v, m0, m1, agg, sem0, sem1):
        c = lax.axis_index("c")
        s = lax.axis_index("s")
        wid = s * NC + c
        base = wid * CPW

        pltpu.sync_copy(dst_hbm.at[pl.ds(base, CPW)], dst_v)
        pltpu.sync_copy(z_hbm.at[pl.ds(0, RPS)], agg.at[pl.ds(s * RPS, RPS)])

        @pl.when(s == 0)
        def _():
            pltpu.sync_copy(z_hbm.at[pl.ds(0, 8)], agg.at[pl.ds(NPAD, 8)])

        bufs = ((m0, sem0), (m1, sem1))

        def start(j, which):
            m, sem = bufs[which]
            pltpu.async_copy(msg_hbm.at[pl.ds((base + j) * CH, CH)], m, sem)

        def finish(j, which):
            m, sem = bufs[which]
            pltpu.make_async_copy(z_hbm.at[pl.ds(0, CH)], m, sem).wait()
            pltpu.sync_copy(m, agg.at[dst_v.at[j]], add=True)

            @pl.when(j + 2 < CPW)
            def _():
                start(j + 2, which)

        start(0, 0)
        start(1, 1)
        plsc.subcore_barrier()

        @pl.loop(0, CPW, step=2)
        def _(j):
            finish(j, 0)
            finish(j + 1, 1)

        plsc.subcore_barrier()
        pltpu.sync_copy(agg.at[pl.ds(s * RPS, RPS)],
                        out_hbm.at[c].at[pl.ds(s * RPS, RPS)])

    return k(msg, dst2d, zeros)


def _tc_edge(attr, w, b):
    def body(a_ref, w_ref, b_ref, o_ref):
        a = a_ref[...]
        w = w_ref[...]
        e = jnp.broadcast_to(b_ref[...], (BE, EMB))
        for kd in range(D_EDGE):
            e = e + a[:, kd:kd + 1] * w[kd:kd + 1, :]
        o_ref[...] = e

    return pl.pallas_call(
        body,
        grid=(E // BE,),
        in_specs=[
            pl.BlockSpec((BE, D_EDGE), lambda i: (i, 0)),
            pl.BlockSpec((D_EDGE, EMB), lambda i: (0, 0)),
            pl.BlockSpec((1, EMB), lambda i: (0, 0)),
        ],
        out_specs=pl.BlockSpec((BE, EMB), lambda i: (i, 0)),
        out_shape=jax.ShapeDtypeStruct((EPAD, EMB), jnp.float32),
    )(attr, w, b)


def _tc_prep(ei_pad):
    BR = 40
    EB = BR * CH

    def body(ei_ref, src_ref, dst_ref):
        i = pl.program_id(0)
        ei = ei_ref[...]
        s2 = ei[0].reshape(BR, CH)
        d2 = ei[1].reshape(BR, CH)
        row = i * BR + lax.broadcasted_iota(jnp.int32, (BR, 1), 0)
        m = row < (E // CH)
        src_ref[...] = jnp.where(m, s2, 0)
        dst_ref[...] = jnp.where(m, d2, NPAD)

    return pl.pallas_call(
        body,
        grid=(NCHUNK // BR,),
        in_specs=[pl.BlockSpec((2, EB), lambda i: (0, i))],
        out_specs=[pl.BlockSpec((BR, CH), lambda i: (i, 0))] * 2,
        out_shape=[jax.ShapeDtypeStruct((NCHUNK, CH), jnp.int32)] * 2,
    )(ei_pad)


def _tc_first(x, node_W, node_b2, batch_row):
    def body(x_ref, w_ref, b_ref, br_ref, hl_ref, seg_ref):
        hl = jnp.dot(x_ref[...], w_ref[...],
                     preferred_element_type=jnp.float32) + b_ref[...]
        hl_ref[...] = hl
        oh_t = (lax.broadcasted_iota(jnp.int32, (G, 1), 0)
                == br_ref[0]).astype(jnp.float32)

        @pl.when(pl.program_id(0) == 0)
        def _():
            seg_ref[...] = jnp.zeros_like(seg_ref)

        seg_ref[...] += jnp.dot(oh_t, hl, preferred_element_type=jnp.float32)

    return pl.pallas_call(
        body,
        grid=(N // BN,),
        in_specs=[
            pl.BlockSpec((BN, EMB), lambda i: (i, 0)),
            pl.BlockSpec((EMB, EMB), lambda i: (0, 0)),
            pl.BlockSpec((1, EMB), lambda i: (0, 0)),
            pl.BlockSpec((1, 1, BN), lambda i: (i, 0, 0)),
        ],
        out_specs=[
            pl.BlockSpec((BN, EMB), lambda i: (i, 0)),
            pl.BlockSpec((G, EMB), lambda i: (0, 0)),
        ],
        out_shape=[
            jax.ShapeDtypeStruct((N, EMB), jnp.float32),
            jax.ShapeDtypeStruct((G, EMB), jnp.float32),
        ],
    )(x, node_W, node_b2, batch_row)


def _tc_mid(h, vn, batch_col, batch_row, want_seg):
    def body(h_ref, vn_ref, bc_ref, br_ref, hl_ref, *rest):
        oh = (bc_ref[...] == lax.broadcasted_iota(jnp.int32, (1, G),
                                                  1)).astype(jnp.float32)
        hl = h_ref[...] + jnp.dot(oh, vn_ref[...],
                                  preferred_element_type=jnp.float32)
        hl_ref[...] = hl
        if want_seg:
            seg_ref = rest[0]
            oh_t = (lax.broadcasted_iota(jnp.int32, (G, 1), 0)
                    == br_ref[0]).astype(jnp.float32)

            @pl.when(pl.program_id(0) == 0)
            def _():
                seg_ref[...] = jnp.zeros_like(seg_ref)

            seg_ref[...] += jnp.dot(oh_t, hl,
                                    preferred_element_type=jnp.float32)

    out_specs = [pl.BlockSpec((BN, EMB), lambda i: (i, 0))]
    out_shape = [jax.ShapeDtypeStruct((N, EMB), jnp.float32)]
    if want_seg:
        out_specs.append(pl.BlockSpec((G, EMB), lambda i: (0, 0)))
        out_shape.append(jax.ShapeDtypeStruct((G, EMB), jnp.float32))
    res = pl.pallas_call(
        body,
        grid=(N // BN,),
        in_specs=[
            pl.BlockSpec((BN, EMB), lambda i: (i, 0)),
            pl.BlockSpec((G, EMB), lambda i: (0, 0)),
            pl.BlockSpec((BN, 1), lambda i: (i, 0)),
            pl.BlockSpec((1, 1, BN), lambda i: (i, 0, 0)),
        ],
        out_specs=out_specs,
        out_shape=out_shape,
    )(h, vn, batch_col, batch_row)
    return res if want_seg else res[0]


def _tc_dense(hl, agg2, eps_l, W1, b1, g1, bb1, W2, b2, g2, bb2, last):
    def body(hl_ref, agg_ref, eps_ref, w1_ref, b1_ref, g1_ref, bb1_ref,
             w2_ref, b2_ref, g2_ref, bb2_ref, o_ref):
        a = agg_ref[0] + agg_ref[1]
        pre = (1.0 + eps_ref[0, 0]) * hl_ref[...] + a
        t = jnp.dot(pre, w1_ref[...],
                    preferred_element_type=jnp.float32) + b1_ref[...]
        t = jnp.maximum(t * g1_ref[...] + bb1_ref[...], 0.0)
        h = jnp.dot(t, w2_ref[...],
                    preferred_element_type=jnp.float32) + b2_ref[...]
        h = h * g2_ref[...] + bb2_ref[...]
        o_ref[...] = h if last else jnp.maximum(h, 0.0)

    return pl.pallas_call(
        body,
        grid=(N // BN,),
        in_specs=[
            pl.BlockSpec((BN, EMB), lambda i: (i, 0)),
            pl.BlockSpec((NC, BN, EMB), lambda i: (0, i, 0)),
            pl.BlockSpec((1, 1), lambda i: (0, 0)),
            pl.BlockSpec((EMB, 2 * EMB), lambda i: (0, 0)),
            pl.BlockSpec((1, 2 * EMB), lambda i: (0, 0)),
            pl.BlockSpec((1, 2 * EMB), lambda i: (0, 0)),
            pl.BlockSpec((1, 2 * EMB), lambda i: (0, 0)),
            pl.BlockSpec((2 * EMB, EMB), lambda i: (0, 0)),
            pl.BlockSpec((1, EMB), lambda i: (0, 0)),
            pl.BlockSpec((1, EMB), lambda i: (0, 0)),
            pl.BlockSpec((1, EMB), lambda i: (0, 0)),
        ],
        out_specs=pl.BlockSpec((BN, EMB), lambda i: (i, 0)),
        out_shape=jax.ShapeDtypeStruct((N, EMB), jnp.float32),
    )(hl, agg2, eps_l, W1, b1, g1, bb1, W2, b2, g2, bb2)


def _tc_vn(seg, vn, W1, b1, g1, bb1, W2, b2, g2, bb2):
    def body(seg_ref, vn_ref, w1_ref, b1_ref, g1_ref, bb1_ref, w2_ref,
             b2_ref, g2_ref, bb2_ref, o_ref):
        v = seg_ref[...] + vn_ref[...]
        u = jnp.dot(v, w1_ref[...],
                    preferred_element_type=jnp.float32) + b1_ref[...]
        u = jnp.maximum(u * g1_ref[...] + bb1_ref[...], 0.0)
        u = jnp.dot(u, w2_ref[...],
                    preferred_element_type=jnp.float32) + b2_ref[...]
        u = u * g2_ref[...] + bb2_ref[...]
        o_ref[...] = jnp.maximum(u, 0.0)

    shapes = [(G, EMB), (G, EMB), (EMB, 2 * EMB), (1, 2 * EMB),
              (1, 2 * EMB), (1, 2 * EMB), (2 * EMB, EMB), (1, EMB),
              (1, EMB), (1, EMB)]
    return pl.pallas_call(
        body,
        in_specs=[pl.BlockSpec(s, lambda: (0, 0)) for s in shapes],
        out_specs=pl.BlockSpec((G, EMB), lambda: (0, 0)),
        out_shape=jax.ShapeDtypeStruct((G, EMB), jnp.float32),
    )(seg, vn, W1, b1, g1, bb1, W2, b2, g2, bb2)


def kernel(x, edge_index, edge_attr, batch, node_W, node_b, eps, edgeW,
           edgeb, mlpW1, mlpb1, bnm_g, bnm_b, mlpW2, mlpb2, bn_g, bn_b,
           vnW1, vnb1, vnbn1_g, vnbn1_b, vnW2, vnb2, vnbn2_g, vnbn2_b):
    src2d, dst2d = _tc_prep(
        jnp.pad(edge_index, ((0, 0), (0, EPAD - E))))
    batch_col = batch.reshape(N, 1)
    batch_row = batch.reshape(N // BN, 1, BN)
    zerosZ = jnp.zeros((RPS, EMB), jnp.float32)

    es = [_tc_edge(edge_attr, edgeW[l], edgeb[l].reshape(1, EMB))
          for l in range(N_LAYER)]

    hl, seg = _tc_first(x, node_W, node_b.reshape(1, EMB), batch_row)
    vn = jnp.zeros((G, EMB), jnp.float32)
    h = None
    for l in range(N_LAYER):
        if l > 0:
            if l < N_LAYER - 1:
                hl, seg = _tc_mid(h, vn, batch_col, batch_row, True)
            else:
                hl = _tc_mid(h, vn, batch_col, batch_row, False)
        msg = _sc_msg(hl, es[l], src2d)
        agg2 = _sc_scatter_add(msg, dst2d, zerosZ)
        h = _tc_dense(hl, agg2, eps[l].reshape(1, 1), mlpW1[l],
                      mlpb1[l].reshape(1, -1), bnm_g[l].reshape(1, -1),
                      bnm_b[l].reshape(1, -1), mlpW2[l],
                      mlpb2[l].reshape(1, -1), bn_g[l].reshape(1, -1),
                      bn_b[l].reshape(1, -1), last=(l == N_LAYER - 1))
        if l < N_LAYER - 1:
            vn = _tc_vn(seg, vn, vnW1[l], vnb1[l].reshape(1, -1),
                        vnbn1_g[l].reshape(1, -1), vnbn1_b[l].reshape(1, -1),
                        vnW2[l], vnb2[l].reshape(1, -1),
                        vnbn2_g[l].reshape(1, -1), vnbn2_b[l].reshape(1, -1))
    return h


# speedup vs baseline: 1.0293x; 1.0293x over previous
import functools

import jax
import jax.numpy as jnp
from jax import lax
from jax.experimental import pallas as pl
from jax.experimental.pallas import tpu as pltpu
from jax.experimental.pallas import tpu_sc as plsc

N_LAYER = 3
EMB = 128
D_EDGE = 4
N = 10000
E = 320000
G = 64

NC, NS = 2, 16
NW = NC * NS
CH = 128
EPAD = 327680
NCHUNK = EPAD // CH
CPW = NCHUNK // NW
NPAD = 10240
RPS = NPAD // NS

BN = 1000
BE = 2048


def _sc_mesh():
    return plsc.VectorSubcoreMesh(core_axis_name="c", subcore_axis_name="s")


def _sc_msg(table, e, src2d):
    TPS = 632
    TLAST = N - (NS - 1) * TPS

    @functools.partial(
        pl.kernel,
        mesh=_sc_mesh(),
        out_type=jax.ShapeDtypeStruct((EPAD, EMB), jnp.float32),
        scratch_types=[
            pltpu.VMEM((4 * CH,), jnp.int32),
            pltpu.VMEM((CH, EMB), jnp.float32),
            pltpu.VMEM((CH, EMB), jnp.float32),
            pltpu.VMEM((CH, EMB), jnp.float32),
            pltpu.VMEM_SHARED((N, EMB), jnp.float32),
            pltpu.SemaphoreType.DMA,
            pltpu.SemaphoreType.DMA,
            pltpu.SemaphoreType.DMA,
            pltpu.SemaphoreType.DMA,
            pltpu.SemaphoreType.DMA,
            pltpu.SemaphoreType.DMA,
            pltpu.SemaphoreType.DMA,
            pltpu.SemaphoreType.DMA,
        ],
    )
    def k(table_hbm, e_hbm, src_hbm, out_hbm, ring, av, e0, e1, tbl,
          semi0, semi1, semi2, semi3, seme0, seme1, semo0, semo1):
        c = lax.axis_index("c")
        s = lax.axis_index("s")
        wid = s * NC + c
        base = wid * CPW
        semis = (semi0, semi1, semi2, semi3)

        toff = pl.multiple_of(s * TPS, 8)

        @pl.when(s < NS - 1)
        def _():
            pltpu.sync_copy(table_hbm.at[pl.ds(toff, TPS)],
                            tbl.at[pl.ds(toff, TPS)])

        @pl.when(s == NS - 1)
        def _():
            pltpu.sync_copy(table_hbm.at[pl.ds((NS - 1) * TPS, TLAST)],
                            tbl.at[pl.ds((NS - 1) * TPS, TLAST)])

        for q in range(4):
            pltpu.async_copy(src_hbm.at[pl.ds((base + q) * CH, CH)],
                             ring.at[pl.ds(q * CH, CH)], semis[q])
        pltpu.async_copy(e_hbm.at[pl.ds(base * CH, CH)], e0, seme0)
        plsc.subcore_barrier()

        bufs = ((e0, seme0, semo0), (e1, seme1, semo1))

        def half(j, q, which):
            eb, seme, semo = bufs[which]
            eo, semeo, semoo = bufs[1 - which]
            semi = semis[q]
            pltpu.make_async_copy(src_hbm.at[pl.ds(0, CH)],
                                  ring.at[pl.ds(0, CH)], semi).wait()
            pltpu.sync_copy(tbl.at[ring.at[pl.ds(q * CH, CH)]], av)
            @pl.when(j + 4 < CPW)
            def _():
                pltpu.async_copy(src_hbm.at[pl.ds((base + j + 4) * CH, CH)],
                                 ring.at[pl.ds(q * CH, CH)], semi)

            @pl.when(j + 1 < CPW)
            def _():
                @pl.when(j >= 1)
                def _():
                    pltpu.make_async_copy(e_hbm.at[pl.ds(0, CH)], eo,
                                          semoo).wait()
                pltpu.async_copy(e_hbm.at[pl.ds((base + j + 1) * CH, CH)],
                                 eo, semeo)

            pltpu.make_async_copy(e_hbm.at[pl.ds(0, CH)], eb, seme).wait()

            @pl.loop(0, CH)
            def _(r):
                for cc in range(EMB // 16):
                    sl = pl.ds(cc * 16, 16)
                    eb[r, sl] = jnp.maximum(av[r, sl] + eb[r, sl], 0.0)

            pltpu.async_copy(eb, out_hbm.at[pl.ds((base + j) * CH, CH)], semo)

        @pl.loop(0, CPW, step=4)
        def _(j):
            half(j, 0, 0)
            half(j + 1, 1, 1)
            half(j + 2, 2, 0)
            half(j + 3, 3, 1)

        pltpu.make_async_copy(e_hbm.at[pl.ds(0, CH)], e0, semo0).wait()
        pltpu.make_async_copy(e_hbm.at[pl.ds(0, CH)], e1, semo1).wait()

    return k(table, e, src2d)


def _sc_scatter_add(msg, dst2d, zeros):
    @functools.partial(
        pl.kernel,
        mesh=_sc_mesh(),
        out_type=jax.ShapeDtypeStruct((NC, NPAD, EMB), jnp.float32),
        scratch_types=[
            pltpu.VMEM((CPW, CH), jnp.int32),
            pltpu.VMEM((CH, EMB), jnp.float32),
            pltpu.VMEM((CH, EMB), jnp.float32),
            pltpu.VMEM_SHARED((NPAD + 8, EMB), jnp.float32),
            pltpu.SemaphoreType.DMA,
            pltpu.SemaphoreType.DMA,
        ],
    )
    def k(msg_hbm, dst_hbm, z_hbm, out_hbm, dst_v, m0, m1, agg, sem0, sem1):
        c = lax.axis_index("c")
        s = lax.axis_index("s")
        wid = s * NC + c
        base = wid * CPW

        pltpu.sync_copy(dst_hbm.at[pl.ds(base, CPW)], dst_v)
        pltpu.sync_copy(z_hbm.at[pl.ds(0, RPS)], agg.at[pl.ds(s * RPS, RPS)])

        @pl.when(s == 0)
        def _():
            pltpu.sync_copy(z_hbm.at[pl.ds(0, 8)], agg.at[pl.ds(NPAD, 8)])

        bufs = ((m0, sem0), (m1, sem1))

        def start(j, which):
            m, sem = bufs[which]
            pltpu.async_copy(msg_hbm.at[pl.ds((base + j) * CH, CH)], m, sem)

        def finish(j, which):
            m, sem = bufs[which]
            pltpu.make_async_copy(z_hbm.at[pl.ds(0, CH)], m, sem).wait()
            pltpu.sync_copy(m, agg.at[dst_v.at[j]], add=True)

            @pl.when(j + 2 < CPW)
            def _():
                start(j + 2, which)

        start(0, 0)
        start(1, 1)
        plsc.subcore_barrier()

        @pl.loop(0, CPW, step=2)
        def _(j):
            finish(j, 0)
            finish(j + 1, 1)

        plsc.subcore_barrier()
        pltpu.sync_copy(agg.at[pl.ds(s * RPS, RPS)],
                        out_hbm.at[c].at[pl.ds(s * RPS, RPS)])

    return k(msg, dst2d, zeros)


def _tc_edge(attr_pad, eW, eb):
    def body(a_ref, w_ref, b_ref, o0_ref, o1_ref, o2_ref):
        a = a_ref[...]
        w = w_ref[...]
        b = b_ref[...]
        outs = (o0_ref, o1_ref, o2_ref)
        for l in range(N_LAYER):
            e = jnp.broadcast_to(b[l:l + 1, :], (BE, EMB))
            for kd in range(D_EDGE):
                e = e + a[:, kd:kd + 1] * w[l * D_EDGE + kd:l * D_EDGE + kd + 1, :]
            outs[l][...] = e

    o = jax.ShapeDtypeStruct((EPAD, EMB), jnp.float32)
    return pl.pallas_call(
        body,
        grid=(EPAD // BE,),
        in_specs=[
            pl.BlockSpec((BE, D_EDGE), lambda i: (i, 0)),
            pl.BlockSpec((N_LAYER * D_EDGE, EMB), lambda i: (0, 0)),
            pl.BlockSpec((N_LAYER, EMB), lambda i: (0, 0)),
        ],
        out_specs=[pl.BlockSpec((BE, EMB), lambda i: (i, 0))] * 3,
        out_shape=[o, o, o],
    )(attr_pad, eW, eb)


def _tc_first(x, node_W, node_b2, batch_row):
    def body(x_ref, w_ref, b_ref, br_ref, hl_ref, seg_ref):
        hl = jnp.dot(x_ref[...], w_ref[...],
                     preferred_element_type=jnp.float32) + b_ref[...]
        hl_ref[...] = hl
        oh_t = (lax.broadcasted_iota(jnp.int32, (G, 1), 0)
                == br_ref[0]).astype(jnp.float32)

        @pl.when(pl.program_id(0) == 0)
        def _():
            seg_ref[...] = jnp.zeros_like(seg_ref)

        seg_ref[...] += jnp.dot(oh_t, hl, preferred_element_type=jnp.float32)

    return pl.pallas_call(
        body,
        grid=(N // BN,),
        in_specs=[
            pl.BlockSpec((BN, EMB), lambda i: (i, 0)),
            pl.BlockSpec((EMB, EMB), lambda i: (0, 0)),
            pl.BlockSpec((1, EMB), lambda i: (0, 0)),
            pl.BlockSpec((1, 1, BN), lambda i: (i, 0, 0)),
        ],
        out_specs=[
            pl.BlockSpec((BN, EMB), lambda i: (i, 0)),
            pl.BlockSpec((G, EMB), lambda i: (0, 0)),
        ],
        out_shape=[
            jax.ShapeDtypeStruct((N, EMB), jnp.float32),
            jax.ShapeDtypeStruct((G, EMB), jnp.float32),
        ],
    )(x, node_W, node_b2, batch_row)


def _tc_mid(h, vn, batch_col, batch_row, want_seg):
    def body(h_ref, vn_ref, bc_ref, br_ref, hl_ref, *rest):
        oh = (bc_ref[...] == lax.broadcasted_iota(jnp.int32, (1, G),
                                                  1)).astype(jnp.float32)
        hl = h_ref[...] + jnp.dot(oh, vn_ref[...],
                                  preferred_element_type=jnp.float32)
        hl_ref[...] = hl
        if want_seg:
            seg_ref = rest[0]
            oh_t = (lax.broadcasted_iota(jnp.int32, (G, 1), 0)
                    == br_ref[0]).astype(jnp.float32)

            @pl.when(pl.program_id(0) == 0)
            def _():
                seg_ref[...] = jnp.zeros_like(seg_ref)

            seg_ref[...] += jnp.dot(oh_t, hl,
                                    preferred_element_type=jnp.float32)

    out_specs = [pl.BlockSpec((BN, EMB), lambda i: (i, 0))]
    out_shape = [jax.ShapeDtypeStruct((N, EMB), jnp.float32)]
    if want_seg:
        out_specs.append(pl.BlockSpec((G, EMB), lambda i: (0, 0)))
        out_shape.append(jax.ShapeDtypeStruct((G, EMB), jnp.float32))
    res = pl.pallas_call(
        body,
        grid=(N // BN,),
        in_specs=[
            pl.BlockSpec((BN, EMB), lambda i: (i, 0)),
            pl.BlockSpec((G, EMB), lambda i: (0, 0)),
            pl.BlockSpec((BN, 1), lambda i: (i, 0)),
            pl.BlockSpec((1, 1, BN), lambda i: (i, 0, 0)),
        ],
        out_specs=out_specs,
        out_shape=out_shape,
    )(h, vn, batch_col, batch_row)
    return res if want_seg else res[0]


def _tc_dense(hl, agg2, eps_l, W1, b1, g1, bb1, W2, b2, g2, bb2, last):
    def body(hl_ref, agg_ref, eps_ref, w1_ref, b1_ref, g1_ref, bb1_ref,
             w2_ref, b2_ref, g2_ref, bb2_ref, o_ref):
        a = agg_ref[0] + agg_ref[1]
        pre = (1.0 + eps_ref[0, 0]) * hl_ref[...] + a
        t = jnp.dot(pre, w1_ref[...],
                    preferred_element_type=jnp.float32) + b1_ref[...]
        t = jnp.maximum(t * g1_ref[...] + bb1_ref[...], 0.0)
        h = jnp.dot(t, w2_ref[...],
                    preferred_element_type=jnp.float32) + b2_ref[...]
        h = h * g2_ref[...] + bb2_ref[...]
        o_ref[...] = h if last else jnp.maximum(h, 0.0)

    return pl.pallas_call(
        body,
        grid=(N // BN,),
        in_specs=[
            pl.BlockSpec((BN, EMB), lambda i: (i, 0)),
            pl.BlockSpec((NC, BN, EMB), lambda i: (0, i, 0)),
            pl.BlockSpec((1, 1), lambda i: (0, 0)),
            pl.BlockSpec((EMB, 2 * EMB), lambda i: (0, 0)),
            pl.BlockSpec((1, 2 * EMB), lambda i: (0, 0)),
            pl.BlockSpec((1, 2 * EMB), lambda i: (0, 0)),
            pl.BlockSpec((1, 2 * EMB), lambda i: (0, 0)),
            pl.BlockSpec((2 * EMB, EMB), lambda i: (0, 0)),
            pl.BlockSpec((1, EMB), lambda i: (0, 0)),
            pl.BlockSpec((1, EMB), lambda i: (0, 0)),
            pl.BlockSpec((1, EMB), lambda i: (0, 0)),
        ],
        out_specs=pl.BlockSpec((BN, EMB), lambda i: (i, 0)),
        out_shape=jax.ShapeDtypeStruct((N, EMB), jnp.float32),
    )(hl, agg2, eps_l, W1, b1, g1, bb1, W2, b2, g2, bb2)


def _tc_vn(seg, vn, W1, b1, g1, bb1, W2, b2, g2, bb2):
    def body(seg_ref, vn_ref, w1_ref, b1_ref, g1_ref, bb1_ref, w2_ref,
             b2_ref, g2_ref, bb2_ref, o_ref):
        v = seg_ref[...] + vn_ref[...]
        u = jnp.dot(v, w1_ref[...],
                    preferred_element_type=jnp.float32) + b1_ref[...]
        u = jnp.maximum(u * g1_ref[...] + bb1_ref[...], 0.0)
        u = jnp.dot(u, w2_ref[...],
                    preferred_element_type=jnp.float32) + b2_ref[...]
        u = u * g2_ref[...] + bb2_ref[...]
        o_ref[...] = jnp.maximum(u, 0.0)

    shapes = [(G, EMB), (G, EMB), (EMB, 2 * EMB), (1, 2 * EMB),
              (1, 2 * EMB), (1, 2 * EMB), (2 * EMB, EMB), (1, EMB),
              (1, EMB), (1, EMB)]
    return pl.pallas_call(
        body,
        in_specs=[pl.BlockSpec(s, lambda: (0, 0)) for s in shapes],
        out_specs=pl.BlockSpec((G, EMB), lambda: (0, 0)),
        out_shape=jax.ShapeDtypeStruct((G, EMB), jnp.float32),
    )(seg, vn, W1, b1, g1, bb1, W2, b2, g2, bb2)


def kernel(x, edge_index, edge_attr, batch, node_W, node_b, eps, edgeW,
           edgeb, mlpW1, mlpb1, bnm_g, bnm_b, mlpW2, mlpb2, bn_g, bn_b,
           vnW1, vnb1, vnbn1_g, vnbn1_b, vnW2, vnb2, vnbn2_g, vnbn2_b):
    src = edge_index[0]
    dst = edge_index[1]
    pad = EPAD - E
    src1d = jnp.concatenate([src, jnp.zeros((pad,), jnp.int32)])
    dst2d = jnp.concatenate(
        [dst, jnp.full((pad,), NPAD, jnp.int32)]).reshape(NCHUNK, CH)
    attr_pad = jnp.concatenate(
        [edge_attr, jnp.zeros((pad, D_EDGE), jnp.float32)], axis=0)
    batch_col = batch.reshape(N, 1)
    batch_row = batch.reshape(N // BN, 1, BN)
    zerosZ = jnp.zeros((RPS, EMB), jnp.float32)

    es = _tc_edge(attr_pad, edgeW.reshape(N_LAYER * D_EDGE, EMB), edgeb)

    hl, seg = _tc_first(x, node_W, node_b.reshape(1, EMB), batch_row)
    vn = jnp.zeros((G, EMB), jnp.float32)
    h = None
    for l in range(N_LAYER):
        if l > 0:
            if l < N_LAYER - 1:
                hl, seg = _tc_mid(h, vn, batch_col, batch_row, True)
            else:
                hl = _tc_mid(h, vn, batch_col, batch_row, False)
        msg = _sc_msg(hl, es[l], src1d)
        agg2 = _sc_scatter_add(msg, dst2d, zerosZ)
        h = _tc_dense(hl, agg2, eps[l].reshape(1, 1), mlpW1[l],
                      mlpb1[l].reshape(1, -1), bnm_g[l].reshape(1, -1),
                      bnm_b[l].reshape(1, -1), mlpW2[l],
                      mlpb2[l].reshape(1, -1), bn_g[l].reshape(1, -1),
                      bn_b[l].reshape(1, -1), last=(l == N_LAYER - 1))
        if l < N_LAYER - 1:
            vn = _tc_vn(seg, vn, vnW1[l], vnb1[l].reshape(1, -1),
                        vnbn1_g[l].reshape(1, -1), vnbn1_b[l].reshape(1, -1),
                        vnW2[l], vnb2[l].reshape(1, -1),
                        vnbn2_g[l].reshape(1, -1), vnbn2_b[l].reshape(1, -1))
    return h


# speedup vs baseline: 1.0664x; 1.0360x over previous
import functools

import jax
import jax.numpy as jnp
from jax import lax
from jax.experimental import pallas as pl
from jax.experimental.pallas import tpu as pltpu
from jax.experimental.pallas import tpu_sc as plsc

N_LAYER = 3
EMB = 128
D_EDGE = 4
N = 10000
E = 320000
G = 64

NC, NS = 2, 16
NW = NC * NS
CH = 128
EPAD = 327680
NCHUNK = EPAD // CH
CPW = NCHUNK // NW
NPAD = 10240
RPS = NPAD // NS

BN = 1000
BE = 4096


def _sc_mesh():
    return plsc.VectorSubcoreMesh(core_axis_name="c", subcore_axis_name="s")


def _sc_msg(table, e, src2d):
    TPS = 632
    TLAST = N - (NS - 1) * TPS

    @functools.partial(
        pl.kernel,
        mesh=_sc_mesh(),
        out_type=jax.ShapeDtypeStruct((EPAD, EMB), jnp.float32),
        scratch_types=[
            pltpu.VMEM((4 * CH,), jnp.int32),
            pltpu.VMEM((CH, EMB), jnp.float32),
            pltpu.VMEM((CH, EMB), jnp.float32),
            pltpu.VMEM((CH, EMB), jnp.float32),
            pltpu.VMEM_SHARED((N, EMB), jnp.float32),
            pltpu.SemaphoreType.DMA,
            pltpu.SemaphoreType.DMA,
            pltpu.SemaphoreType.DMA,
            pltpu.SemaphoreType.DMA,
            pltpu.SemaphoreType.DMA,
            pltpu.SemaphoreType.DMA,
            pltpu.SemaphoreType.DMA,
            pltpu.SemaphoreType.DMA,
        ],
    )
    def k(table_hbm, e_hbm, src_hbm, out_hbm, ring, av, e0, e1, tbl,
          semi0, semi1, semi2, semi3, seme0, seme1, semo0, semo1):
        c = lax.axis_index("c")
        s = lax.axis_index("s")
        wid = s * NC + c
        base = wid * CPW
        semis = (semi0, semi1, semi2, semi3)

        toff = pl.multiple_of(s * TPS, 8)

        @pl.when(s < NS - 1)
        def _():
            pltpu.sync_copy(table_hbm.at[pl.ds(toff, TPS)],
                            tbl.at[pl.ds(toff, TPS)])

        @pl.when(s == NS - 1)
        def _():
            pltpu.sync_copy(table_hbm.at[pl.ds((NS - 1) * TPS, TLAST)],
                            tbl.at[pl.ds((NS - 1) * TPS, TLAST)])

        for q in range(4):
            pltpu.async_copy(src_hbm.at[pl.ds((base + q) * CH, CH)],
                             ring.at[pl.ds(q * CH, CH)], semis[q])
        pltpu.async_copy(e_hbm.at[pl.ds(base * CH, CH)], e0, seme0)
        plsc.subcore_barrier()

        bufs = ((e0, seme0, semo0), (e1, seme1, semo1))

        def half(j, q, which):
            eb, seme, semo = bufs[which]
            eo, semeo, semoo = bufs[1 - which]
            semi = semis[q]
            pltpu.make_async_copy(src_hbm.at[pl.ds(0, CH)],
                                  ring.at[pl.ds(0, CH)], semi).wait()
            pltpu.sync_copy(tbl.at[ring.at[pl.ds(q * CH, CH)]], av)
            @pl.when(j + 4 < CPW)
            def _():
                pltpu.async_copy(src_hbm.at[pl.ds((base + j + 4) * CH, CH)],
                                 ring.at[pl.ds(q * CH, CH)], semi)

            @pl.when(j + 1 < CPW)
            def _():
                @pl.when(j >= 1)
                def _():
                    pltpu.make_async_copy(e_hbm.at[pl.ds(0, CH)], eo,
                                          semoo).wait()
                pltpu.async_copy(e_hbm.at[pl.ds((base + j + 1) * CH, CH)],
                                 eo, semeo)

            pltpu.make_async_copy(e_hbm.at[pl.ds(0, CH)], eb, seme).wait()

            @pl.loop(0, CH)
            def _(r):
                for cc in range(EMB // 16):
                    sl = pl.ds(cc * 16, 16)
                    eb[r, sl] = jnp.maximum(av[r, sl] + eb[r, sl], 0.0)

            pltpu.async_copy(eb, out_hbm.at[pl.ds((base + j) * CH, CH)], semo)

        @pl.loop(0, CPW, step=4)
        def _(j):
            half(j, 0, 0)
            half(j + 1, 1, 1)
            half(j + 2, 2, 0)
            half(j + 3, 3, 1)

        pltpu.make_async_copy(e_hbm.at[pl.ds(0, CH)], e0, semo0).wait()
        pltpu.make_async_copy(e_hbm.at[pl.ds(0, CH)], e1, semo1).wait()

    return k(table, e, src2d)


def _sc_scatter_add(msg, dst2d, zeros):
    @functools.partial(
        pl.kernel,
        mesh=_sc_mesh(),
        out_type=jax.ShapeDtypeStruct((NC, NPAD, EMB), jnp.float32),
        scratch_types=[
            pltpu.VMEM((CPW, CH), jnp.int32),
            pltpu.VMEM((CH, EMB), jnp.float32),
            pltpu.VMEM((CH, EMB), jnp.float32),
            pltpu.VMEM_SHARED((NPAD + 8, EMB), jnp.float32),
            pltpu.SemaphoreType.DMA,
            pltpu.SemaphoreType.DMA,
        ],
    )
    def k(msg_hbm, dst_hbm, z_hbm, out_hbm, dst_v, m0, m1, agg, sem0, sem1):
        c = lax.axis_index("c")
        s = lax.axis_index("s")
        wid = s * NC + c
        base = wid * CPW

        pltpu.sync_copy(dst_hbm.at[pl.ds(base, CPW)], dst_v)
        pltpu.sync_copy(z_hbm.at[pl.ds(0, RPS)], agg.at[pl.ds(s * RPS, RPS)])

        @pl.when(s == 0)
        def _():
            pltpu.sync_copy(z_hbm.at[pl.ds(0, 8)], agg.at[pl.ds(NPAD, 8)])

        bufs = ((m0, sem0), (m1, sem1))

        def start(j, which):
            m, sem = bufs[which]
            pltpu.async_copy(msg_hbm.at[pl.ds((base + j) * CH, CH)], m, sem)

        def finish(j, which):
            m, sem = bufs[which]
            pltpu.make_async_copy(z_hbm.at[pl.ds(0, CH)], m, sem).wait()
            pltpu.sync_copy(m, agg.at[dst_v.at[j]], add=True)

            @pl.when(j + 2 < CPW)
            def _():
                start(j + 2, which)

        start(0, 0)
        start(1, 1)
        plsc.subcore_barrier()

        @pl.loop(0, CPW, step=2)
        def _(j):
            finish(j, 0)
            finish(j + 1, 1)

        plsc.subcore_barrier()
        pltpu.sync_copy(agg.at[pl.ds(s * RPS, RPS)],
                        out_hbm.at[c].at[pl.ds(s * RPS, RPS)])

    return k(msg, dst2d, zeros)


def _tc_edge(attr_pad, eW, eb):
    def body(a_ref, w_ref, b_ref, o0_ref, o1_ref, o2_ref):
        a = a_ref[...]
        w = w_ref[...]
        b = b_ref[...]
        outs = (o0_ref, o1_ref, o2_ref)
        for l in range(N_LAYER):
            e = jnp.broadcast_to(b[l:l + 1, :], (BE, EMB))
            for kd in range(D_EDGE):
                e = e + a[:, kd:kd + 1] * w[l * D_EDGE + kd:l * D_EDGE + kd + 1, :]
            outs[l][...] = e

    o = jax.ShapeDtypeStruct((EPAD, EMB), jnp.float32)
    return pl.pallas_call(
        body,
        grid=(EPAD // BE,),
        in_specs=[
            pl.BlockSpec((BE, D_EDGE), lambda i: (i, 0)),
            pl.BlockSpec((N_LAYER * D_EDGE, EMB), lambda i: (0, 0)),
            pl.BlockSpec((N_LAYER, EMB), lambda i: (0, 0)),
        ],
        out_specs=[pl.BlockSpec((BE, EMB), lambda i: (i, 0))] * 3,
        out_shape=[o, o, o],
    )(attr_pad, eW, eb)


def _tc_first(x, node_W, node_b2, batch_row):
    def body(x_ref, w_ref, b_ref, br_ref, hl_ref, seg_ref):
        hl = jnp.dot(x_ref[...], w_ref[...],
                     preferred_element_type=jnp.float32) + b_ref[...]
        hl_ref[...] = hl
        oh_t = (lax.broadcasted_iota(jnp.int32, (G, 1), 0)
                == br_ref[0]).astype(jnp.float32)

        @pl.when(pl.program_id(0) == 0)
        def _():
            seg_ref[...] = jnp.zeros_like(seg_ref)

        seg_ref[...] += jnp.dot(oh_t, hl, preferred_element_type=jnp.float32)

    return pl.pallas_call(
        body,
        grid=(N // BN,),
        in_specs=[
            pl.BlockSpec((BN, EMB), lambda i: (i, 0)),
            pl.BlockSpec((EMB, EMB), lambda i: (0, 0)),
            pl.BlockSpec((1, EMB), lambda i: (0, 0)),
            pl.BlockSpec((1, 1, BN), lambda i: (i, 0, 0)),
        ],
        out_specs=[
            pl.BlockSpec((BN, EMB), lambda i: (i, 0)),
            pl.BlockSpec((G, EMB), lambda i: (0, 0)),
        ],
        out_shape=[
            jax.ShapeDtypeStruct((N, EMB), jnp.float32),
            jax.ShapeDtypeStruct((G, EMB), jnp.float32),
        ],
    )(x, node_W, node_b2, batch_row)


def _tc_mid(h, vn, batch_col, batch_row, want_seg):
    def body(h_ref, vn_ref, bc_ref, br_ref, hl_ref, *rest):
        oh = (bc_ref[...] == lax.broadcasted_iota(jnp.int32, (1, G),
                                                  1)).astype(jnp.float32)
        hl = h_ref[...] + jnp.dot(oh, vn_ref[...],
                                  preferred_element_type=jnp.float32)
        hl_ref[...] = hl
        if want_seg:
            seg_ref = rest[0]
            oh_t = (lax.broadcasted_iota(jnp.int32, (G, 1), 0)
                    == br_ref[0]).astype(jnp.float32)

            @pl.when(pl.program_id(0) == 0)
            def _():
                seg_ref[...] = jnp.zeros_like(seg_ref)

            seg_ref[...] += jnp.dot(oh_t, hl,
                                    preferred_element_type=jnp.float32)

    out_specs = [pl.BlockSpec((BN, EMB), lambda i: (i, 0))]
    out_shape = [jax.ShapeDtypeStruct((N, EMB), jnp.float32)]
    if want_seg:
        out_specs.append(pl.BlockSpec((G, EMB), lambda i: (0, 0)))
        out_shape.append(jax.ShapeDtypeStruct((G, EMB), jnp.float32))
    res = pl.pallas_call(
        body,
        grid=(N // BN,),
        in_specs=[
            pl.BlockSpec((BN, EMB), lambda i: (i, 0)),
            pl.BlockSpec((G, EMB), lambda i: (0, 0)),
            pl.BlockSpec((BN, 1), lambda i: (i, 0)),
            pl.BlockSpec((1, 1, BN), lambda i: (i, 0, 0)),
        ],
        out_specs=out_specs,
        out_shape=out_shape,
    )(h, vn, batch_col, batch_row)
    return res if want_seg else res[0]


def _tc_dense(hl, agg2, eps_l, W1, b1, g1, bb1, W2, b2, g2, bb2, last):
    def body(hl_ref, agg_ref, eps_ref, w1_ref, b1_ref, g1_ref, bb1_ref,
             w2_ref, b2_ref, g2_ref, bb2_ref, o_ref):
        a = agg_ref[0] + agg_ref[1]
        pre = (1.0 + eps_ref[0, 0]) * hl_ref[...] + a
        t = jnp.dot(pre, w1_ref[...],
                    preferred_element_type=jnp.float32) + b1_ref[...]
        t = jnp.maximum(t * g1_ref[...] + bb1_ref[...], 0.0)
        h = jnp.dot(t, w2_ref[...],
                    preferred_element_type=jnp.float32) + b2_ref[...]
        h = h * g2_ref[...] + bb2_ref[...]
        o_ref[...] = h if last else jnp.maximum(h, 0.0)

    return pl.pallas_call(
        body,
        grid=(N // BN,),
        in_specs=[
            pl.BlockSpec((BN, EMB), lambda i: (i, 0)),
            pl.BlockSpec((NC, BN, EMB), lambda i: (0, i, 0)),
            pl.BlockSpec((1, 1), lambda i: (0, 0)),
            pl.BlockSpec((EMB, 2 * EMB), lambda i: (0, 0)),
            pl.BlockSpec((1, 2 * EMB), lambda i: (0, 0)),
            pl.BlockSpec((1, 2 * EMB), lambda i: (0, 0)),
            pl.BlockSpec((1, 2 * EMB), lambda i: (0, 0)),
            pl.BlockSpec((2 * EMB, EMB), lambda i: (0, 0)),
            pl.BlockSpec((1, EMB), lambda i: (0, 0)),
            pl.BlockSpec((1, EMB), lambda i: (0, 0)),
            pl.BlockSpec((1, EMB), lambda i: (0, 0)),
        ],
        out_specs=pl.BlockSpec((BN, EMB), lambda i: (i, 0)),
        out_shape=jax.ShapeDtypeStruct((N, EMB), jnp.float32),
    )(hl, agg2, eps_l, W1, b1, g1, bb1, W2, b2, g2, bb2)


def _tc_vn(seg, vn, W1, b1, g1, bb1, W2, b2, g2, bb2):
    def body(seg_ref, vn_ref, w1_ref, b1_ref, g1_ref, bb1_ref, w2_ref,
             b2_ref, g2_ref, bb2_ref, o_ref):
        v = seg_ref[...] + vn_ref[...]
        u = jnp.dot(v, w1_ref[...],
                    preferred_element_type=jnp.float32) + b1_ref[...]
        u = jnp.maximum(u * g1_ref[...] + bb1_ref[...], 0.0)
        u = jnp.dot(u, w2_ref[...],
                    preferred_element_type=jnp.float32) + b2_ref[...]
        u = u * g2_ref[...] + bb2_ref[...]
        o_ref[...] = jnp.maximum(u, 0.0)

    shapes = [(G, EMB), (G, EMB), (EMB, 2 * EMB), (1, 2 * EMB),
              (1, 2 * EMB), (1, 2 * EMB), (2 * EMB, EMB), (1, EMB),
              (1, EMB), (1, EMB)]
    return pl.pallas_call(
        body,
        in_specs=[pl.BlockSpec(s, lambda: (0, 0)) for s in shapes],
        out_specs=pl.BlockSpec((G, EMB), lambda: (0, 0)),
        out_shape=jax.ShapeDtypeStruct((G, EMB), jnp.float32),
    )(seg, vn, W1, b1, g1, bb1, W2, b2, g2, bb2)


def kernel(x, edge_index, edge_attr, batch, node_W, node_b, eps, edgeW,
           edgeb, mlpW1, mlpb1, bnm_g, bnm_b, mlpW2, mlpb2, bn_g, bn_b,
           vnW1, vnb1, vnbn1_g, vnbn1_b, vnW2, vnb2, vnbn2_g, vnbn2_b):
    src = edge_index[0]
    dst = edge_index[1]
    pad = EPAD - E
    src1d = jnp.concatenate([src, jnp.zeros((pad,), jnp.int32)])
    dst2d = jnp.concatenate(
        [dst, jnp.full((pad,), NPAD, jnp.int32)]).reshape(NCHUNK, CH)
    attr_pad = jnp.concatenate(
        [edge_attr, jnp.zeros((pad, D_EDGE), jnp.float32)], axis=0)
    batch_col = batch.reshape(N, 1)
    batch_row = batch.reshape(N // BN, 1, BN)
    zerosZ = jnp.zeros((RPS, EMB), jnp.float32)

    es = _tc_edge(attr_pad, edgeW.reshape(N_LAYER * D_EDGE, EMB), edgeb)

    hl, seg = _tc_first(x, node_W, node_b.reshape(1, EMB), batch_row)
    vn = jnp.zeros((G, EMB), jnp.float32)
    h = None
    for l in range(N_LAYER):
        if l > 0:
            if l < N_LAYER - 1:
                hl, seg = _tc_mid(h, vn, batch_col, batch_row, True)
            else:
                hl = _tc_mid(h, vn, batch_col, batch_row, False)
        msg = _sc_msg(hl, es[l], src1d)
        agg2 = _sc_scatter_add(msg, dst2d, zerosZ)
        h = _tc_dense(hl, agg2, eps[l].reshape(1, 1), mlpW1[l],
                      mlpb1[l].reshape(1, -1), bnm_g[l].reshape(1, -1),
                      bnm_b[l].reshape(1, -1), mlpW2[l],
                      mlpb2[l].reshape(1, -1), bn_g[l].reshape(1, -1),
                      bn_b[l].reshape(1, -1), last=(l == N_LAYER - 1))
        if l < N_LAYER - 1:
            vn = _tc_vn(seg, vn, vnW1[l], vnb1[l].reshape(1, -1),
                        vnbn1_g[l].reshape(1, -1), vnbn1_b[l].reshape(1, -1),
                        vnW2[l], vnb2[l].reshape(1, -1),
                        vnbn2_g[l].reshape(1, -1), vnbn2_b[l].reshape(1, -1))
    return h


# speedup vs baseline: 1.0791x; 1.0119x over previous
import functools

import jax
import jax.numpy as jnp
from jax import lax
from jax.experimental import pallas as pl
from jax.experimental.pallas import tpu as pltpu
from jax.experimental.pallas import tpu_sc as plsc

N_LAYER = 3
EMB = 128
D_EDGE = 4
N = 10000
E = 320000
G = 64

NC, NS = 2, 16
NW = NC * NS
CH = 128
EPAD = 327680
NCHUNK = EPAD // CH
CPW = NCHUNK // NW
NPAD = 10240
RPS = NPAD // NS

BN = 1000
BE = 8192


def _sc_mesh():
    return plsc.VectorSubcoreMesh(core_axis_name="c", subcore_axis_name="s")


def _sc_msg(table, e, src2d):
    TPS = 632
    TLAST = N - (NS - 1) * TPS

    @functools.partial(
        pl.kernel,
        mesh=_sc_mesh(),
        out_type=jax.ShapeDtypeStruct((EPAD, EMB), jnp.float32),
        scratch_types=[
            pltpu.VMEM((4 * CH,), jnp.int32),
            pltpu.VMEM((CH, EMB), jnp.float32),
            pltpu.VMEM((CH, EMB), jnp.float32),
            pltpu.VMEM((CH, EMB), jnp.float32),
            pltpu.VMEM_SHARED((N, EMB), jnp.float32),
            pltpu.SemaphoreType.DMA,
            pltpu.SemaphoreType.DMA,
            pltpu.SemaphoreType.DMA,
            pltpu.SemaphoreType.DMA,
            pltpu.SemaphoreType.DMA,
            pltpu.SemaphoreType.DMA,
            pltpu.SemaphoreType.DMA,
            pltpu.SemaphoreType.DMA,
        ],
    )
    def k(table_hbm, e_hbm, src_hbm, out_hbm, ring, av, e0, e1, tbl,
          semi0, semi1, semi2, semi3, seme0, seme1, semo0, semo1):
        c = lax.axis_index("c")
        s = lax.axis_index("s")
        wid = s * NC + c
        base = wid * CPW
        semis = (semi0, semi1, semi2, semi3)

        toff = pl.multiple_of(s * TPS, 8)

        @pl.when(s < NS - 1)
        def _():
            pltpu.sync_copy(table_hbm.at[pl.ds(toff, TPS)],
                            tbl.at[pl.ds(toff, TPS)])

        @pl.when(s == NS - 1)
        def _():
            pltpu.sync_copy(table_hbm.at[pl.ds((NS - 1) * TPS, TLAST)],
                            tbl.at[pl.ds((NS - 1) * TPS, TLAST)])

        for q in range(4):
            pltpu.async_copy(src_hbm.at[pl.ds((base + q) * CH, CH)],
                             ring.at[pl.ds(q * CH, CH)], semis[q])
        pltpu.async_copy(e_hbm.at[pl.ds(base * CH, CH)], e0, seme0)
        plsc.subcore_barrier()

        bufs = ((e0, seme0, semo0), (e1, seme1, semo1))

        def half(j, q, which):
            eb, seme, semo = bufs[which]
            eo, semeo, semoo = bufs[1 - which]
            semi = semis[q]
            pltpu.make_async_copy(src_hbm.at[pl.ds(0, CH)],
                                  ring.at[pl.ds(0, CH)], semi).wait()
            pltpu.sync_copy(tbl.at[ring.at[pl.ds(q * CH, CH)]], av)
            @pl.when(j + 4 < CPW)
            def _():
                pltpu.async_copy(src_hbm.at[pl.ds((base + j + 4) * CH, CH)],
                                 ring.at[pl.ds(q * CH, CH)], semi)

            @pl.when(j + 1 < CPW)
            def _():
                @pl.when(j >= 1)
                def _():
                    pltpu.make_async_copy(e_hbm.at[pl.ds(0, CH)], eo,
                                          semoo).wait()
                pltpu.async_copy(e_hbm.at[pl.ds((base + j + 1) * CH, CH)],
                                 eo, semeo)

            pltpu.make_async_copy(e_hbm.at[pl.ds(0, CH)], eb, seme).wait()

            @pl.loop(0, CH)
            def _(r):
                for cc in range(EMB // 16):
                    sl = pl.ds(cc * 16, 16)
                    eb[r, sl] = jnp.maximum(av[r, sl] + eb[r, sl], 0.0)

            pltpu.async_copy(eb, out_hbm.at[pl.ds((base + j) * CH, CH)], semo)

        @pl.loop(0, CPW, step=4)
        def _(j):
            half(j, 0, 0)
            half(j + 1, 1, 1)
            half(j + 2, 2, 0)
            half(j + 3, 3, 1)

        pltpu.make_async_copy(e_hbm.at[pl.ds(0, CH)], e0, semo0).wait()
        pltpu.make_async_copy(e_hbm.at[pl.ds(0, CH)], e1, semo1).wait()

    return k(table, e, src2d)


def _sc_scatter_add(msg, dst2d, zeros):
    @functools.partial(
        pl.kernel,
        mesh=_sc_mesh(),
        out_type=jax.ShapeDtypeStruct((NC, NPAD, EMB), jnp.float32),
        scratch_types=[
            pltpu.VMEM((CPW, CH), jnp.int32),
            pltpu.VMEM((CH, EMB), jnp.float32),
            pltpu.VMEM((CH, EMB), jnp.float32),
            pltpu.VMEM_SHARED((NPAD + 8, EMB), jnp.float32),
            pltpu.SemaphoreType.DMA,
            pltpu.SemaphoreType.DMA,
        ],
    )
    def k(msg_hbm, dst_hbm, z_hbm, out_hbm, dst_v, m0, m1, agg, sem0, sem1):
        c = lax.axis_index("c")
        s = lax.axis_index("s")
        wid = s * NC + c
        base = wid * CPW

        pltpu.sync_copy(dst_hbm.at[pl.ds(base, CPW)], dst_v)
        pltpu.sync_copy(z_hbm.at[pl.ds(0, RPS)], agg.at[pl.ds(s * RPS, RPS)])

        @pl.when(s == 0)
        def _():
            pltpu.sync_copy(z_hbm.at[pl.ds(0, 8)], agg.at[pl.ds(NPAD, 8)])

        bufs = ((m0, sem0), (m1, sem1))

        def start(j, which):
            m, sem = bufs[which]
            pltpu.async_copy(msg_hbm.at[pl.ds((base + j) * CH, CH)], m, sem)

        def finish(j, which):
            m, sem = bufs[which]
            pltpu.make_async_copy(z_hbm.at[pl.ds(0, CH)], m, sem).wait()
            pltpu.sync_copy(m, agg.at[dst_v.at[j]], add=True)

            @pl.when(j + 2 < CPW)
            def _():
                start(j + 2, which)

        start(0, 0)
        start(1, 1)
        plsc.subcore_barrier()

        @pl.loop(0, CPW, step=2)
        def _(j):
            finish(j, 0)
            finish(j + 1, 1)

        plsc.subcore_barrier()
        pltpu.sync_copy(agg.at[pl.ds(s * RPS, RPS)],
                        out_hbm.at[c].at[pl.ds(s * RPS, RPS)])

    return k(msg, dst2d, zeros)


def _tc_edge(attr_pad, eW, eb):
    def body(a_ref, w_ref, b_ref, o0_ref, o1_ref, o2_ref):
        a = a_ref[...]
        w = w_ref[...]
        b = b_ref[...]
        outs = (o0_ref, o1_ref, o2_ref)
        for l in range(N_LAYER):
            e = jnp.broadcast_to(b[l:l + 1, :], (BE, EMB))
            for kd in range(D_EDGE):
                e = e + a[:, kd:kd + 1] * w[l * D_EDGE + kd:l * D_EDGE + kd + 1, :]
            outs[l][...] = e

    o = jax.ShapeDtypeStruct((EPAD, EMB), jnp.float32)
    return pl.pallas_call(
        body,
        grid=(EPAD // BE,),
        in_specs=[
            pl.BlockSpec((BE, D_EDGE), lambda i: (i, 0)),
            pl.BlockSpec((N_LAYER * D_EDGE, EMB), lambda i: (0, 0)),
            pl.BlockSpec((N_LAYER, EMB), lambda i: (0, 0)),
        ],
        out_specs=[pl.BlockSpec((BE, EMB), lambda i: (i, 0))] * 3,
        out_shape=[o, o, o],
    )(attr_pad, eW, eb)


def _tc_first(x, node_W, node_b2, batch_row):
    def body(x_ref, w_ref, b_ref, br_ref, hl_ref, seg_ref):
        hl = jnp.dot(x_ref[...], w_ref[...],
                     preferred_element_type=jnp.float32) + b_ref[...]
        hl_ref[...] = hl
        oh_t = (lax.broadcasted_iota(jnp.int32, (G, 1), 0)
                == br_ref[0]).astype(jnp.float32)

        @pl.when(pl.program_id(0) == 0)
        def _():
            seg_ref[...] = jnp.zeros_like(seg_ref)

        seg_ref[...] += jnp.dot(oh_t, hl, preferred_element_type=jnp.float32)

    return pl.pallas_call(
        body,
        grid=(N // BN,),
        in_specs=[
            pl.BlockSpec((BN, EMB), lambda i: (i, 0)),
            pl.BlockSpec((EMB, EMB), lambda i: (0, 0)),
            pl.BlockSpec((1, EMB), lambda i: (0, 0)),
            pl.BlockSpec((1, 1, BN), lambda i: (i, 0, 0)),
        ],
        out_specs=[
            pl.BlockSpec((BN, EMB), lambda i: (i, 0)),
            pl.BlockSpec((G, EMB), lambda i: (0, 0)),
        ],
        out_shape=[
            jax.ShapeDtypeStruct((N, EMB), jnp.float32),
            jax.ShapeDtypeStruct((G, EMB), jnp.float32),
        ],
    )(x, node_W, node_b2, batch_row)


def _tc_mid(h, vn, batch_col, batch_row, want_seg):
    def body(h_ref, vn_ref, bc_ref, br_ref, hl_ref, *rest):
        oh = (bc_ref[...] == lax.broadcasted_iota(jnp.int32, (1, G),
                                                  1)).astype(jnp.float32)
        hl = h_ref[...] + jnp.dot(oh, vn_ref[...],
                                  preferred_element_type=jnp.float32)
        hl_ref[...] = hl
        if want_seg:
            seg_ref = rest[0]
            oh_t = (lax.broadcasted_iota(jnp.int32, (G, 1), 0)
                    == br_ref[0]).astype(jnp.float32)

            @pl.when(pl.program_id(0) == 0)
            def _():
                seg_ref[...] = jnp.zeros_like(seg_ref)

            seg_ref[...] += jnp.dot(oh_t, hl,
                                    preferred_element_type=jnp.float32)

    out_specs = [pl.BlockSpec((BN, EMB), lambda i: (i, 0))]
    out_shape = [jax.ShapeDtypeStruct((N, EMB), jnp.float32)]
    if want_seg:
        out_specs.append(pl.BlockSpec((G, EMB), lambda i: (0, 0)))
        out_shape.append(jax.ShapeDtypeStruct((G, EMB), jnp.float32))
    res = pl.pallas_call(
        body,
        grid=(N // BN,),
        in_specs=[
            pl.BlockSpec((BN, EMB), lambda i: (i, 0)),
            pl.BlockSpec((G, EMB), lambda i: (0, 0)),
            pl.BlockSpec((BN, 1), lambda i: (i, 0)),
            pl.BlockSpec((1, 1, BN), lambda i: (i, 0, 0)),
        ],
        out_specs=out_specs,
        out_shape=out_shape,
    )(h, vn, batch_col, batch_row)
    return res if want_seg else res[0]


def _tc_dense(hl, agg2, eps_l, W1, b1, g1, bb1, W2, b2, g2, bb2, last):
    def body(hl_ref, agg_ref, eps_ref, w1_ref, b1_ref, g1_ref, bb1_ref,
             w2_ref, b2_ref, g2_ref, bb2_ref, o_ref):
        a = agg_ref[0] + agg_ref[1]
        pre = (1.0 + eps_ref[0, 0]) * hl_ref[...] + a
        t = jnp.dot(pre, w1_ref[...],
                    preferred_element_type=jnp.float32) + b1_ref[...]
        t = jnp.maximum(t * g1_ref[...] + bb1_ref[...], 0.0)
        h = jnp.dot(t, w2_ref[...],
                    preferred_element_type=jnp.float32) + b2_ref[...]
        h = h * g2_ref[...] + bb2_ref[...]
        o_ref[...] = h if last else jnp.maximum(h, 0.0)

    return pl.pallas_call(
        body,
        grid=(N // BN,),
        in_specs=[
            pl.BlockSpec((BN, EMB), lambda i: (i, 0)),
            pl.BlockSpec((NC, BN, EMB), lambda i: (0, i, 0)),
            pl.BlockSpec((1, 1), lambda i: (0, 0)),
            pl.BlockSpec((EMB, 2 * EMB), lambda i: (0, 0)),
            pl.BlockSpec((1, 2 * EMB), lambda i: (0, 0)),
            pl.BlockSpec((1, 2 * EMB), lambda i: (0, 0)),
            pl.BlockSpec((1, 2 * EMB), lambda i: (0, 0)),
            pl.BlockSpec((2 * EMB, EMB), lambda i: (0, 0)),
            pl.BlockSpec((1, EMB), lambda i: (0, 0)),
            pl.BlockSpec((1, EMB), lambda i: (0, 0)),
            pl.BlockSpec((1, EMB), lambda i: (0, 0)),
        ],
        out_specs=pl.BlockSpec((BN, EMB), lambda i: (i, 0)),
        out_shape=jax.ShapeDtypeStruct((N, EMB), jnp.float32),
    )(hl, agg2, eps_l, W1, b1, g1, bb1, W2, b2, g2, bb2)


def _tc_vn(seg, vn, W1, b1, g1, bb1, W2, b2, g2, bb2):
    def body(seg_ref, vn_ref, w1_ref, b1_ref, g1_ref, bb1_ref, w2_ref,
             b2_ref, g2_ref, bb2_ref, o_ref):
        v = seg_ref[...] + vn_ref[...]
        u = jnp.dot(v, w1_ref[...],
                    preferred_element_type=jnp.float32) + b1_ref[...]
        u = jnp.maximum(u * g1_ref[...] + bb1_ref[...], 0.0)
        u = jnp.dot(u, w2_ref[...],
                    preferred_element_type=jnp.float32) + b2_ref[...]
        u = u * g2_ref[...] + bb2_ref[...]
        o_ref[...] = jnp.maximum(u, 0.0)

    shapes = [(G, EMB), (G, EMB), (EMB, 2 * EMB), (1, 2 * EMB),
              (1, 2 * EMB), (1, 2 * EMB), (2 * EMB, EMB), (1, EMB),
              (1, EMB), (1, EMB)]
    return pl.pallas_call(
        body,
        in_specs=[pl.BlockSpec(s, lambda: (0, 0)) for s in shapes],
        out_specs=pl.BlockSpec((G, EMB), lambda: (0, 0)),
        out_shape=jax.ShapeDtypeStruct((G, EMB), jnp.float32),
    )(seg, vn, W1, b1, g1, bb1, W2, b2, g2, bb2)


def kernel(x, edge_index, edge_attr, batch, node_W, node_b, eps, edgeW,
           edgeb, mlpW1, mlpb1, bnm_g, bnm_b, mlpW2, mlpb2, bn_g, bn_b,
           vnW1, vnb1, vnbn1_g, vnbn1_b, vnW2, vnb2, vnbn2_g, vnbn2_b):
    src = edge_index[0]
    dst = edge_index[1]
    pad = EPAD - E
    src1d = jnp.concatenate([src, jnp.zeros((pad,), jnp.int32)])
    dst2d = jnp.concatenate(
        [dst, jnp.full((pad,), NPAD, jnp.int32)]).reshape(NCHUNK, CH)
    attr_pad = jnp.concatenate(
        [edge_attr, jnp.zeros((pad, D_EDGE), jnp.float32)], axis=0)
    batch_col = batch.reshape(N, 1)
    batch_row = batch.reshape(N // BN, 1, BN)
    zerosZ = jnp.zeros((RPS, EMB), jnp.float32)

    es = _tc_edge(attr_pad, edgeW.reshape(N_LAYER * D_EDGE, EMB), edgeb)

    hl, seg = _tc_first(x, node_W, node_b.reshape(1, EMB), batch_row)
    vn = jnp.zeros((G, EMB), jnp.float32)
    h = None
    for l in range(N_LAYER):
        if l > 0:
            if l < N_LAYER - 1:
                hl, seg = _tc_mid(h, vn, batch_col, batch_row, True)
            else:
                hl = _tc_mid(h, vn, batch_col, batch_row, False)
        msg = _sc_msg(hl, es[l], src1d)
        agg2 = _sc_scatter_add(msg, dst2d, zerosZ)
        h = _tc_dense(hl, agg2, eps[l].reshape(1, 1), mlpW1[l],
                      mlpb1[l].reshape(1, -1), bnm_g[l].reshape(1, -1),
                      bnm_b[l].reshape(1, -1), mlpW2[l],
                      mlpb2[l].reshape(1, -1), bn_g[l].reshape(1, -1),
                      bn_b[l].reshape(1, -1), last=(l == N_LAYER - 1))
        if l < N_LAYER - 1:
            vn = _tc_vn(seg, vn, vnW1[l], vnb1[l].reshape(1, -1),
                        vnbn1_g[l].reshape(1, -1), vnbn1_b[l].reshape(1, -1),
                        vnW2[l], vnb2[l].reshape(1, -1),
                        vnbn2_g[l].reshape(1, -1), vnbn2_b[l].reshape(1, -1))
    return h
